# baseline (speedup 1.0000x reference)
import functools

import jax
import jax.numpy as jnp
from jax import lax
from jax.experimental import pallas as pl
from jax.experimental.pallas import tpu as pltpu
from jax.experimental.pallas import tpu_sc as plsc

L = 16
G = 128
NB = 4
NW = 32


def kernel(hash_ids, offsets, table):
  B, S, H = hash_ids.shape
  V, D = table.shape
  assert H == L
  N = B * S * H
  R = N // NW
  NG = R // G
  assert R % G == 0 and NG % NB == 0

  mesh = plsc.VectorSubcoreMesh(core_axis_name="c", subcore_axis_name="s")

  @functools.partial(
      pl.kernel,
      out_type=jax.ShapeDtypeStruct((N, D), table.dtype),
      mesh=mesh,
      scratch_types=[
          pltpu.VMEM((R,), jnp.int32),
          pltpu.VMEM((L,), jnp.int32),
          pltpu.VMEM((NG, G), jnp.int32),
          *[pltpu.VMEM((G, D), jnp.float32) for _ in range(NB)],
          *[pltpu.SemaphoreType.DMA for _ in range(NB)],
      ],
      compiler_params=pltpu.CompilerParams(use_tc_tiling_on_sc=False),
  )
  def run(hash_hbm, off_hbm, table_hbm, out_hbm, hash_v, off_v, idx_v, *rest):
    rows = rest[:NB]
    sems = rest[NB:]
    wid = lax.axis_index("s") * 2 + lax.axis_index("c")
    base = wid * R
    pltpu.sync_copy(off_hbm, off_v)
    pltpu.sync_copy(hash_hbm.at[pl.ds(base, R)], hash_v)
    off = off_v[...]

    def fire(g, b):
      for k in range(G // L):
        idx_v[g, pl.ds(k * L, L)] = hash_v[pl.ds(g * G + k * L, L)] + off
      pltpu.async_copy(table_hbm.at[idx_v.at[g]], rows[b], sems[b])

    def drain(g, b):
      pltpu.make_async_copy(table_hbm.at[idx_v.at[g]], rows[b], sems[b]).wait()
      pltpu.sync_copy(rows[b], out_hbm.at[pl.ds(base + g * G, G)])

    for b in range(NB):
      fire(b, b)

    @pl.loop(0, NG // NB - 1)
    def body(outer):
      for b in range(NB):
        g = outer * NB + b
        drain(g, b)
        fire(g + NB, b)

    for b in range(NB):
      drain(NG - NB + b, b)

  out = run(hash_ids.reshape(N), offsets, table)
  return out.reshape(B, S, H, D)
